# baseline (speedup 1.0000x reference)
import functools

import jax
import jax.numpy as jnp
from jax import lax
from jax.experimental import pallas as pl
from jax.experimental.pallas import tpu as pltpu
from jax.experimental.pallas import tpu_sc as plsc

N_LORS_TOTAL = 3 * 65536
N_SAMPLES = 32
EPS = 1e-8
KW2 = 3.0 * 3.0 * 3.141592653589793
SCALE2 = 1.0 / (31.0 * 31.0 * KW2)

NC, NS = 2, 16
HALF = 1024 * 1024
DUMP = 2048
ACC_SIZE = HALF + DUMP

B = 32
ENT = B * N_SAMPLES * 8
LORS_PER_TILE = N_LORS_TOTAL // NS
NBLK = LORS_PER_TILE // B

_mesh = plsc.VectorSubcoreMesh(core_axis_name="c", subcore_axis_name="s")


def _body(img_hbm, eff_hbm, lors_hbm, out_hbm,
          lorA, lorB, scaleA, scaleB, idxA, idxB, wA, wB, valsA, valsB,
          svalbuf, semA, semB, acc_sp):
    core = lax.axis_index("c")
    sid = lax.axis_index("s")
    zero16 = jnp.zeros((16,), jnp.float32)

    def pass1(blk, lorbuf, scalebuf, idxbuf, wbuf):
        pltpu.sync_copy(lors_hbm.at[pl.ds(blk * (6 * B), 6 * B)], lorbuf)
        for i16 in range(B // 16):
            sl = lambda r: pl.ds(r * B + i16 * 16, 16)
            p1x = lorbuf[sl(0)]
            p1y = lorbuf[sl(1)]
            p1z = lorbuf[sl(2)]
            dx = lorbuf[sl(3)] - p1x
            dy = lorbuf[sl(4)] - p1y
            dz = lorbuf[sl(5)] - p1z
            n2 = dx * dx + dy * dy + dz * dz
            scalebuf[pl.ds(i16 * 16, 16)] = n2 * SCALE2
            ax = p1x * 0.5 + 63.5
            ay = p1y * 0.5 + 63.5
            az = p1z * 0.5 + 63.5
            hx = dx * 0.5
            hy = dy * 0.5
            hz = dz * 0.5

            def samp(s, _):
                t = s.astype(jnp.float32) * (1.0 / 31.0)
                cx = ax + hx * t
                cy = ay + hy * t
                cz = az + hz * t
                bx = cx.astype(jnp.int32)
                by = cy.astype(jnp.int32)
                bz = cz.astype(jnp.int32)
                fx = cx - bx.astype(jnp.float32)
                fy = cy - by.astype(jnp.float32)
                fz = cz - bz.astype(jnp.float32)
                gx = 1.0 - fx
                gy = 1.0 - fy
                gz = 1.0 - fz
                ibase = bx * 16384 + by * 128 + bz
                ggx = gx * gy
                gfx = gx * fy
                fgx = fx * gy
                ffx = fx * fy
                row = i16 * N_SAMPLES + s
                corners = (
                    (ibase, ggx * gz), (ibase + 1, ggx * fz),
                    (ibase + 128, gfx * gz), (ibase + 129, gfx * fz),
                    (ibase + 16384, fgx * gz), (ibase + 16385, fgx * fz),
                    (ibase + 16512, ffx * gz), (ibase + 16513, ffx * fz),
                )
                for c, (iv, wv) in enumerate(corners):
                    cs = pl.ds(row * 128 + c * 16, 16)
                    idxbuf[cs] = iv
                    wbuf[cs] = wv
                return 0
            lax.fori_loop(0, N_SAMPLES, samp, 0)

    def gstart(idxbuf, valsbuf, sem):
        pltpu.async_copy(img_hbm.at[idxbuf], valsbuf, sem)

    def gwait(idxbuf, valsbuf, sem):
        pltpu.make_async_copy(img_hbm.at[idxbuf], valsbuf, sem).wait()

    def pass2(scalebuf, idxbuf, wbuf, valsbuf):
        for i16 in range(B // 16):
            def red(s, a16):
                row = i16 * N_SAMPLES + s
                for c in range(8):
                    cs = pl.ds(row * 128 + c * 16, 16)
                    a16 = a16 + valsbuf[cs] * wbuf[cs]
                return a16
            acc16 = lax.fori_loop(0, N_SAMPLES, red, zero16)
            q16 = acc16 * scalebuf[pl.ds(i16 * 16, 16)]

            def sval(s, _):
                row = i16 * N_SAMPLES + s
                for c in range(8):
                    cs = pl.ds(row * 128 + c * 16, 16)
                    svalbuf[cs] = wbuf[cs] * q16
                    idx = idxbuf[cs]
                    own = lax.shift_right_logical(idx, 20) == core
                    lidx = jnp.where(own, idx - core * HALF,
                                     HALF + (idx & (DUMP - 1)))
                    idxbuf[cs] = lidx
                return 0
            lax.fori_loop(0, N_SAMPLES, sval, 0)
        pltpu.sync_copy(svalbuf, acc_sp.at[idxbuf], add=True)

    def zb(i, _):
        svalbuf[pl.ds(i * 16, 16)] = zero16
        return 0
    lax.fori_loop(0, 512, zb, 0)
    for j in range(8):
        pltpu.sync_copy(svalbuf.at[pl.ds(0, 8192)],
                        acc_sp.at[pl.ds(sid * 65536 + j * 8192, 8192)])
    plsc.subcore_barrier()

    base = sid * NBLK
    pass1(base, lorA, scaleA, idxA, wA)
    gstart(idxA, valsA, semA)

    def pipe(i, _):
        b1 = base + 2 * i + 1
        pass1(b1, lorB, scaleB, idxB, wB)
        gstart(idxB, valsB, semB)
        gwait(idxA, valsA, semA)
        pass2(scaleA, idxA, wA, valsA)

        @pl.when(2 * i + 2 < NBLK)
        def _():
            pass1(base + 2 * i + 2, lorA, scaleA, idxA, wA)
            gstart(idxA, valsA, semA)

        gwait(idxB, valsB, semB)
        pass2(scaleB, idxB, wB, valsB)
        return 0
    lax.fori_loop(0, NBLK // 2, pipe, 0)
    plsc.subcore_barrier()

    def finj(j, _):
        loff = sid * 65536 + j * 4096
        goff = core * HALF + loff
        accv = svalbuf.at[pl.ds(0, 4096)]
        imgv = valsA.at[pl.ds(0, 4096)]
        effv = valsB.at[pl.ds(0, 4096)]
        pltpu.sync_copy(acc_sp.at[pl.ds(loff, 4096)], accv)
        pltpu.sync_copy(img_hbm.at[pl.ds(goff, 4096)], imgv)
        pltpu.sync_copy(eff_hbm.at[pl.ds(goff, 4096)], effv)

        def fin(i, _):
            sl = pl.ds(i * 16, 16)
            svalbuf[sl] = valsA[sl] / (valsB[sl] + EPS) * svalbuf[sl]
            return 0
        lax.fori_loop(0, 256, fin, 0)
        pltpu.sync_copy(accv, out_hbm.at[pl.ds(goff, 4096)])
        return 0
    lax.fori_loop(0, 16, finj, 0)


_sc_call = functools.partial(
    pl.kernel,
    out_type=jax.ShapeDtypeStruct((128 * 128 * 128,), jnp.float32),
    mesh=_mesh,
    scratch_types=[
        pltpu.VMEM((6 * B,), jnp.float32),
        pltpu.VMEM((6 * B,), jnp.float32),
        pltpu.VMEM((B,), jnp.float32),
        pltpu.VMEM((B,), jnp.float32),
        pltpu.VMEM((ENT,), jnp.int32),
        pltpu.VMEM((ENT,), jnp.int32),
        pltpu.VMEM((ENT,), jnp.float32),
        pltpu.VMEM((ENT,), jnp.float32),
        pltpu.VMEM((ENT,), jnp.float32),
        pltpu.VMEM((ENT,), jnp.float32),
        pltpu.VMEM((ENT,), jnp.float32),
        pltpu.SemaphoreType.DMA,
        pltpu.SemaphoreType.DMA,
        pltpu.VMEM_SHARED((ACC_SIZE,), jnp.float32),
    ],
)(_body)


def kernel(image, efficiency_map, xlors, ylors, zlors):
    perm = jnp.array([2, 0, 1, 5, 3, 4], dtype=jnp.int32)
    lors = jnp.concatenate([zlors, xlors[:, perm], ylors[:, perm]], axis=0)
    lors_blocked = lors.T.reshape(6, N_LORS_TOTAL // B, B).transpose(1, 0, 2)
    out = _sc_call(image.reshape(-1), efficiency_map.reshape(-1),
                   lors_blocked.reshape(-1))
    return out.reshape(128, 128, 128)


# speedup vs baseline: 1.2498x; 1.2498x over previous
import functools

import jax
import jax.numpy as jnp
from jax import lax
from jax.experimental import pallas as pl
from jax.experimental.pallas import tpu as pltpu
from jax.experimental.pallas import tpu_sc as plsc

N_LORS_TOTAL = 3 * 65536
N_SAMPLES = 32
EPS = 1e-8
KW2 = 3.0 * 3.0 * 3.141592653589793
SCALE2 = 1.0 / (31.0 * 31.0 * KW2)

NC, NS = 2, 16
HALF = 1024 * 1024
DUMP = 2048
ACC_SIZE = HALF + DUMP + 256

B = 32
GROUPS = (B // 16) * N_SAMPLES
GENT = GROUPS * 64
SENT = GROUPS * 128
LORS_PER_TILE = N_LORS_TOTAL // NS
NBLK = LORS_PER_TILE // B

_mesh = plsc.VectorSubcoreMesh(core_axis_name="c", subcore_axis_name="s")


def _body(img2_hbm, img_hbm, eff_hbm, lors_hbm, out_hbm,
          lorA, lorB, scaleA, scaleB, gidxA, gidxB, wxyA, wxyB,
          fzA, fzB, ibA, ibB, valsA, valsB, sidxbuf, svalbuf, finv,
          semA, semB, acc_sp):
    core = lax.axis_index("c")
    sid = lax.axis_index("s")
    zero16 = jnp.zeros((16,), jnp.float32)

    def pass1(blk, lorbuf, scalebuf, gidxbuf, wxybuf, fzbuf, ibbuf):
        pltpu.sync_copy(lors_hbm.at[pl.ds(blk * (6 * B), 6 * B)], lorbuf)
        for i16 in range(B // 16):
            sl = lambda r: pl.ds(r * B + i16 * 16, 16)
            p1x = lorbuf[sl(0)]
            p1y = lorbuf[sl(1)]
            p1z = lorbuf[sl(2)]
            dx = lorbuf[sl(3)] - p1x
            dy = lorbuf[sl(4)] - p1y
            dz = lorbuf[sl(5)] - p1z
            n2 = dx * dx + dy * dy + dz * dz
            scalebuf[pl.ds(i16 * 16, 16)] = n2 * SCALE2
            ax = p1x * 0.5 + 63.5
            ay = p1y * 0.5 + 63.5
            az = p1z * 0.5 + 63.5
            hx = dx * 0.5
            hy = dy * 0.5
            hz = dz * 0.5

            def samp(s, _):
                t = s.astype(jnp.float32) * (1.0 / 31.0)
                cx = ax + hx * t
                cy = ay + hy * t
                cz = az + hz * t
                bx = cx.astype(jnp.int32)
                by = cy.astype(jnp.int32)
                bz = cz.astype(jnp.int32)
                fx = cx - bx.astype(jnp.float32)
                fy = cy - by.astype(jnp.float32)
                fz = cz - bz.astype(jnp.float32)
                gx = 1.0 - fx
                gy = 1.0 - fy
                ibase = bx * 16384 + by * 128 + bz
                which = ibase & 1
                grow0 = lax.shift_right_logical(ibase, 1) + which * HALF
                g = i16 * N_SAMPLES + s
                gidxbuf[pl.ds(g * 64, 16)] = grow0
                gidxbuf[pl.ds(g * 64 + 16, 16)] = grow0 + 64
                gidxbuf[pl.ds(g * 64 + 32, 16)] = grow0 + 8192
                gidxbuf[pl.ds(g * 64 + 48, 16)] = grow0 + 8256
                wxybuf[pl.ds(g * 64, 16)] = gx * gy
                wxybuf[pl.ds(g * 64 + 16, 16)] = gx * fy
                wxybuf[pl.ds(g * 64 + 32, 16)] = fx * gy
                wxybuf[pl.ds(g * 64 + 48, 16)] = fx * fy
                fzbuf[pl.ds(g * 16, 16)] = fz
                ibbuf[pl.ds(g * 16, 16)] = ibase
                return 0
            lax.fori_loop(0, N_SAMPLES, samp, 0)

    def gstart(gidxbuf, valsbuf, sem):
        pltpu.async_copy(img2_hbm.at[gidxbuf], valsbuf, sem)

    def gwait(gidxbuf, valsbuf, sem):
        pltpu.make_async_copy(img2_hbm.at[gidxbuf], valsbuf, sem).wait()

    def pass2(scalebuf, gidxbuf, wxybuf, fzbuf, ibbuf, valsbuf):
        for i16 in range(B // 16):
            def red(s, a16):
                g = i16 * N_SAMPLES + s
                fz = fzbuf[pl.ds(g * 16, 16)]
                gz = 1.0 - fz
                for p in range(4):
                    pw = valsbuf[pl.ds(g * 64 + p * 16, 16)]
                    v0 = plsc.bitcast(lax.shift_left(pw, 16), jnp.float32)
                    v1 = plsc.bitcast(pw & jnp.int32(-65536), jnp.float32)
                    w = wxybuf[pl.ds(g * 64 + p * 16, 16)]
                    a16 = a16 + w * (v0 * gz + v1 * fz)
                return a16
            acc16 = lax.fori_loop(0, N_SAMPLES, red, zero16)
            q16 = acc16 * scalebuf[pl.ds(i16 * 16, 16)]

            def sval(s, _):
                g = i16 * N_SAMPLES + s
                fz = fzbuf[pl.ds(g * 16, 16)]
                gz = 1.0 - fz
                ib = ibbuf[pl.ds(g * 16, 16)]
                own0 = lax.shift_right_logical(ib, 20) == core
                l0 = jnp.where(own0, ib - core * HALF,
                               HALF + (ib & (DUMP - 1)))
                ib1 = ib + 16384
                own1 = lax.shift_right_logical(ib1, 20) == core
                l1 = jnp.where(own1, ib1 - core * HALF,
                               HALF + (ib1 & (DUMP - 1)))
                for p, (lx, yoff) in enumerate(
                        ((l0, 0), (l0, 128), (l1, 0), (l1, 128))):
                    a = wxybuf[pl.ds(g * 64 + p * 16, 16)] * q16
                    e = g * 128 + p * 32
                    svalbuf[pl.ds(e, 16)] = a * gz
                    svalbuf[pl.ds(e + 16, 16)] = a * fz
                    sidxbuf[pl.ds(e, 16)] = lx + yoff
                    sidxbuf[pl.ds(e + 16, 16)] = lx + (yoff + 1)
                return 0
            lax.fori_loop(0, N_SAMPLES, sval, 0)
        pltpu.sync_copy(svalbuf, acc_sp.at[sidxbuf], add=True)

    def zb(i, _):
        svalbuf[pl.ds(i * 16, 16)] = zero16
        return 0
    lax.fori_loop(0, 512, zb, 0)
    def zc(j, _):
        pltpu.sync_copy(svalbuf.at[pl.ds(0, 8192)],
                        acc_sp.at[pl.ds(sid * 65536 + j * 8192, 8192)])
        return 0
    lax.fori_loop(0, 8, zc, 0)
    plsc.subcore_barrier()

    base = sid * NBLK
    pass1(base, lorA, scaleA, gidxA, wxyA, fzA, ibA)
    gstart(gidxA, valsA, semA)

    def pipe(i, _):
        b1 = base + 2 * i + 1
        pass1(b1, lorB, scaleB, gidxB, wxyB, fzB, ibB)
        gstart(gidxB, valsB, semB)
        gwait(gidxA, valsA, semA)
        pass2(scaleA, gidxA, wxyA, fzA, ibA, valsA)

        @pl.when(2 * i + 2 < NBLK)
        def _():
            pass1(base + 2 * i + 2, lorA, scaleA, gidxA, wxyA, fzA, ibA)
            gstart(gidxA, valsA, semA)

        gwait(gidxB, valsB, semB)
        pass2(scaleB, gidxB, wxyB, fzB, ibB, valsB)
        return 0
    lax.fori_loop(0, NBLK // 2, pipe, 0)
    plsc.subcore_barrier()

    def finj(j, _):
        loff = sid * 65536 + j * 4096
        goff = core * HALF + loff
        accv = svalbuf.at[pl.ds(0, 4096)]
        imgv = svalbuf.at[pl.ds(4096, 4096)]
        effv = finv
        pltpu.sync_copy(acc_sp.at[pl.ds(loff, 4096)], accv)
        pltpu.sync_copy(img_hbm.at[pl.ds(goff, 4096)], imgv)
        pltpu.sync_copy(eff_hbm.at[pl.ds(goff, 4096)], effv)

        def fin(i, _):
            sl = pl.ds(i * 16, 16)
            sle = pl.ds(4096 + i * 16, 16)
            svalbuf[sl] = svalbuf[sle] / (finv[sl] + EPS) * svalbuf[sl]
            return 0
        lax.fori_loop(0, 256, fin, 0)
        pltpu.sync_copy(accv, out_hbm.at[pl.ds(goff, 4096)])
        return 0
    lax.fori_loop(0, 16, finj, 0)


_sc_call = functools.partial(
    pl.kernel,
    out_type=jax.ShapeDtypeStruct((128 * 128 * 128,), jnp.float32),
    mesh=_mesh,
    compiler_params=pltpu.CompilerParams(needs_layout_passes=False),
    scratch_types=[
        pltpu.VMEM((6 * B,), jnp.float32),
        pltpu.VMEM((6 * B,), jnp.float32),
        pltpu.VMEM((B,), jnp.float32),
        pltpu.VMEM((B,), jnp.float32),
        pltpu.VMEM((GENT,), jnp.int32),
        pltpu.VMEM((GENT,), jnp.int32),
        pltpu.VMEM((GENT,), jnp.float32),
        pltpu.VMEM((GENT,), jnp.float32),
        pltpu.VMEM((GROUPS * 16,), jnp.float32),
        pltpu.VMEM((GROUPS * 16,), jnp.float32),
        pltpu.VMEM((GROUPS * 16,), jnp.int32),
        pltpu.VMEM((GROUPS * 16,), jnp.int32),
        pltpu.VMEM((GENT,), jnp.int32),
        pltpu.VMEM((GENT,), jnp.int32),
        pltpu.VMEM((SENT,), jnp.int32),
        pltpu.VMEM((SENT,), jnp.float32),
        pltpu.VMEM((4096,), jnp.float32),
        pltpu.SemaphoreType.DMA,
        pltpu.SemaphoreType.DMA,
        pltpu.VMEM_SHARED((ACC_SIZE,), jnp.float32),
    ],
)(_body)


def kernel(image, efficiency_map, xlors, ylors, zlors):
    perm = jnp.array([2, 0, 1, 5, 3, 4], dtype=jnp.int32)
    lors = jnp.concatenate([zlors, xlors[:, perm], ylors[:, perm]], axis=0)
    lors_blocked = lors.T.reshape(6, N_LORS_TOTAL // B, B).transpose(1, 0, 2)
    img_flat = image.reshape(-1)
    bits = jax.lax.bitcast_convert_type(img_flat.astype(jnp.bfloat16),
                                        jnp.uint16).astype(jnp.uint32)
    bsh = jnp.concatenate([bits[1:], bits[:1]])
    wordsA = bits[0::2] | (bits[1::2] << 16)
    wordsB = bsh[0::2] | (bsh[1::2] << 16)
    img2 = jax.lax.bitcast_convert_type(
        jnp.concatenate([wordsA, wordsB]), jnp.int32)
    out = _sc_call(img2, img_flat, efficiency_map.reshape(-1),
                   lors_blocked.reshape(-1))
    return out.reshape(128, 128, 128)


# speedup vs baseline: 1.4331x; 1.1466x over previous
import functools

import jax
import jax.numpy as jnp
from jax import lax
from jax.experimental import pallas as pl
from jax.experimental.pallas import tpu as pltpu
from jax.experimental.pallas import tpu_sc as plsc

N_LORS_TOTAL = 3 * 65536
N_SAMPLES = 32
EPS = 1e-8
KW2 = 3.0 * 3.0 * 3.141592653589793
SCALE2 = 1.0 / (31.0 * 31.0 * KW2)

NC, NS = 2, 16
HALF = 1024 * 1024
DUMP = 2048
ACC_SIZE = HALF + DUMP + 256

B = 32
GROUPS = (B // 16) * N_SAMPLES
GENT = GROUPS * 64
SENT = GROUPS * 128
LORS_PER_TILE = N_LORS_TOTAL // NS
NBLK = LORS_PER_TILE // B

_mesh = plsc.VectorSubcoreMesh(core_axis_name="c", subcore_axis_name="s")


def _body(img2_hbm, img_hbm, eff_hbm, lors_hbm, out_hbm,
          lorA, lorB, scaleA, scaleB, gidxA, gidxB, wxyA, wxyB,
          fzA, fzB, ibA, ibB, valsA, valsB, sidxA, sidxB, svalA, svalB,
          semA, semB, ssemA, ssemB, acc_sp):
    core = lax.axis_index("c")
    sid = lax.axis_index("s")
    zero16 = jnp.zeros((16,), jnp.float32)

    def pass1(blk, lorbuf, scalebuf, gidxbuf, wxybuf, fzbuf, ibbuf):
        pltpu.sync_copy(lors_hbm.at[pl.ds(blk * (6 * B), 6 * B)], lorbuf)
        for i16 in range(B // 16):
            sl = lambda r: pl.ds(r * B + i16 * 16, 16)
            p1x = lorbuf[sl(0)]
            p1y = lorbuf[sl(1)]
            p1z = lorbuf[sl(2)]
            dx = lorbuf[sl(3)] - p1x
            dy = lorbuf[sl(4)] - p1y
            dz = lorbuf[sl(5)] - p1z
            n2 = dx * dx + dy * dy + dz * dz
            scalebuf[pl.ds(i16 * 16, 16)] = n2 * SCALE2
            ax = p1x * 0.5 + 63.5
            ay = p1y * 0.5 + 63.5
            az = p1z * 0.5 + 63.5
            hx = dx * 0.5
            hy = dy * 0.5
            hz = dz * 0.5

            def samp(s, _):
                t = s.astype(jnp.float32) * (1.0 / 31.0)
                cx = ax + hx * t
                cy = ay + hy * t
                cz = az + hz * t
                bx = cx.astype(jnp.int32)
                by = cy.astype(jnp.int32)
                bz = cz.astype(jnp.int32)
                fx = cx - bx.astype(jnp.float32)
                fy = cy - by.astype(jnp.float32)
                fz = cz - bz.astype(jnp.float32)
                gx = 1.0 - fx
                gy = 1.0 - fy
                ibase = bx * 16384 + by * 128 + bz
                which = ibase & 1
                grow0 = lax.shift_right_logical(ibase, 1) + which * HALF
                g = i16 * N_SAMPLES + s
                gidxbuf[pl.ds(g * 64, 16)] = grow0
                gidxbuf[pl.ds(g * 64 + 16, 16)] = grow0 + 64
                gidxbuf[pl.ds(g * 64 + 32, 16)] = grow0 + 8192
                gidxbuf[pl.ds(g * 64 + 48, 16)] = grow0 + 8256
                wxybuf[pl.ds(g * 64, 16)] = gx * gy
                wxybuf[pl.ds(g * 64 + 16, 16)] = gx * fy
                wxybuf[pl.ds(g * 64 + 32, 16)] = fx * gy
                wxybuf[pl.ds(g * 64 + 48, 16)] = fx * fy
                fzbuf[pl.ds(g * 16, 16)] = fz
                ibbuf[pl.ds(g * 16, 16)] = ibase
                return 0
            lax.fori_loop(0, N_SAMPLES, samp, 0, unroll=4)

    def gstart(gidxbuf, valsbuf, sem):
        pltpu.async_copy(img2_hbm.at[gidxbuf], valsbuf, sem)

    def gwait(gidxbuf, valsbuf, sem):
        pltpu.make_async_copy(img2_hbm.at[gidxbuf], valsbuf, sem).wait()

    def swait(sidxbuf, svalbuf, ssem):
        pltpu.make_async_copy(svalbuf, acc_sp.at[sidxbuf], ssem).wait()

    def pass2(scalebuf, gidxbuf, wxybuf, fzbuf, ibbuf, valsbuf,
              sidxbuf, svalbuf, ssem):
        for i16 in range(B // 16):
            def red(s, a16):
                g = i16 * N_SAMPLES + s
                fz = fzbuf[pl.ds(g * 16, 16)]
                gz = 1.0 - fz
                for p in range(4):
                    pw = valsbuf[pl.ds(g * 64 + p * 16, 16)]
                    v0 = plsc.bitcast(lax.shift_left(pw, 16), jnp.float32)
                    v1 = plsc.bitcast(pw & jnp.int32(-65536), jnp.float32)
                    w = wxybuf[pl.ds(g * 64 + p * 16, 16)]
                    a16 = a16 + w * (v0 * gz + v1 * fz)
                return a16
            acc16 = lax.fori_loop(0, N_SAMPLES, red, zero16, unroll=4)
            q16 = acc16 * scalebuf[pl.ds(i16 * 16, 16)]

            def sval(s, _):
                g = i16 * N_SAMPLES + s
                fz = fzbuf[pl.ds(g * 16, 16)]
                gz = 1.0 - fz
                ib = ibbuf[pl.ds(g * 16, 16)]
                own0 = lax.shift_right_logical(ib, 20) == core
                l0 = jnp.where(own0, ib - core * HALF,
                               HALF + (ib & (DUMP - 1)))
                ib1 = ib + 16384
                own1 = lax.shift_right_logical(ib1, 20) == core
                l1 = jnp.where(own1, ib1 - core * HALF,
                               HALF + (ib1 & (DUMP - 1)))
                for p, (lx, yoff) in enumerate(
                        ((l0, 0), (l0, 128), (l1, 0), (l1, 128))):
                    a = wxybuf[pl.ds(g * 64 + p * 16, 16)] * q16
                    e = g * 128 + p * 32
                    svalbuf[pl.ds(e, 16)] = a * gz
                    svalbuf[pl.ds(e + 16, 16)] = a * fz
                    sidxbuf[pl.ds(e, 16)] = lx + yoff
                    sidxbuf[pl.ds(e + 16, 16)] = lx + (yoff + 1)
                return 0
            lax.fori_loop(0, N_SAMPLES, sval, 0, unroll=4)
        pltpu.async_copy(svalbuf, acc_sp.at[sidxbuf], sem=ssem, add=True)

    def zb(i, _):
        svalA[pl.ds(i * 16, 16)] = zero16
        return 0
    lax.fori_loop(0, 512, zb, 0)
    def zc(j, _):
        pltpu.sync_copy(svalA.at[pl.ds(0, 8192)],
                        acc_sp.at[pl.ds(sid * 65536 + j * 8192, 8192)])
        return 0
    lax.fori_loop(0, 8, zc, 0)
    plsc.subcore_barrier()

    base = sid * NBLK
    pass1(base, lorA, scaleA, gidxA, wxyA, fzA, ibA)
    gstart(gidxA, valsA, semA)

    def pipe(i, _):
        b1 = base + 2 * i + 1
        pass1(b1, lorB, scaleB, gidxB, wxyB, fzB, ibB)
        gstart(gidxB, valsB, semB)
        gwait(gidxA, valsA, semA)

        @pl.when(i > 0)
        def _():
            swait(sidxA, svalA, ssemA)

        pass2(scaleA, gidxA, wxyA, fzA, ibA, valsA, sidxA, svalA, ssemA)

        @pl.when(2 * i + 2 < NBLK)
        def _():
            pass1(base + 2 * i + 2, lorA, scaleA, gidxA, wxyA, fzA, ibA)
            gstart(gidxA, valsA, semA)

        gwait(gidxB, valsB, semB)

        @pl.when(i > 0)
        def _():
            swait(sidxB, svalB, ssemB)

        pass2(scaleB, gidxB, wxyB, fzB, ibB, valsB, sidxB, svalB, ssemB)
        return 0
    lax.fori_loop(0, NBLK // 2, pipe, 0)
    swait(sidxA, svalA, ssemA)
    swait(sidxB, svalB, ssemB)
    plsc.subcore_barrier()

    def finj(j, _):
        loff = sid * 65536 + j * 2048
        goff = core * HALF + loff
        accv = svalA.at[pl.ds(0, 2048)]
        imgv = svalA.at[pl.ds(2048, 2048)]
        effv = svalA.at[pl.ds(4096, 2048)]
        pltpu.sync_copy(acc_sp.at[pl.ds(loff, 2048)], accv)
        pltpu.sync_copy(img_hbm.at[pl.ds(goff, 2048)], imgv)
        pltpu.sync_copy(eff_hbm.at[pl.ds(goff, 2048)], effv)

        def fin(i, _):
            sl = pl.ds(i * 16, 16)
            sli = pl.ds(2048 + i * 16, 16)
            sle = pl.ds(4096 + i * 16, 16)
            svalA[sl] = svalA[sli] / (svalA[sle] + EPS) * svalA[sl]
            return 0
        lax.fori_loop(0, 128, fin, 0)
        pltpu.sync_copy(accv, out_hbm.at[pl.ds(goff, 2048)])
        return 0
    lax.fori_loop(0, 32, finj, 0)


_sc_call = functools.partial(
    pl.kernel,
    out_type=jax.ShapeDtypeStruct((128 * 128 * 128,), jnp.float32),
    mesh=_mesh,
    compiler_params=pltpu.CompilerParams(needs_layout_passes=False),
    scratch_types=[
        pltpu.VMEM((6 * B,), jnp.float32),
        pltpu.VMEM((6 * B,), jnp.float32),
        pltpu.VMEM((B,), jnp.float32),
        pltpu.VMEM((B,), jnp.float32),
        pltpu.VMEM((GENT,), jnp.int32),
        pltpu.VMEM((GENT,), jnp.int32),
        pltpu.VMEM((GENT,), jnp.float32),
        pltpu.VMEM((GENT,), jnp.float32),
        pltpu.VMEM((GROUPS * 16,), jnp.float32),
        pltpu.VMEM((GROUPS * 16,), jnp.float32),
        pltpu.VMEM((GROUPS * 16,), jnp.int32),
        pltpu.VMEM((GROUPS * 16,), jnp.int32),
        pltpu.VMEM((GENT,), jnp.int32),
        pltpu.VMEM((GENT,), jnp.int32),
        pltpu.VMEM((SENT,), jnp.int32),
        pltpu.VMEM((SENT,), jnp.int32),
        pltpu.VMEM((SENT,), jnp.float32),
        pltpu.VMEM((SENT,), jnp.float32),
        pltpu.SemaphoreType.DMA,
        pltpu.SemaphoreType.DMA,
        pltpu.SemaphoreType.DMA,
        pltpu.SemaphoreType.DMA,
        pltpu.VMEM_SHARED((ACC_SIZE,), jnp.float32),
    ],
)(_body)


def kernel(image, efficiency_map, xlors, ylors, zlors):
    perm = jnp.array([2, 0, 1, 5, 3, 4], dtype=jnp.int32)
    lors = jnp.concatenate([zlors, xlors[:, perm], ylors[:, perm]], axis=0)
    lors_blocked = lors.T.reshape(6, N_LORS_TOTAL // B, B).transpose(1, 0, 2)
    img_flat = image.reshape(-1)
    bits = jax.lax.bitcast_convert_type(img_flat.astype(jnp.bfloat16),
                                        jnp.uint16).astype(jnp.uint32)
    bsh = jnp.concatenate([bits[1:], bits[:1]])
    wordsA = bits[0::2] | (bits[1::2] << 16)
    wordsB = bsh[0::2] | (bsh[1::2] << 16)
    img2 = jax.lax.bitcast_convert_type(
        jnp.concatenate([wordsA, wordsB]), jnp.int32)
    out = _sc_call(img2, img_flat, efficiency_map.reshape(-1),
                   lors_blocked.reshape(-1))
    return out.reshape(128, 128, 128)
